# baseline (speedup 1.0000x reference)
import functools

import jax
import jax.numpy as jnp
from jax import lax
from jax.experimental import pallas as pl
from jax.experimental.pallas import tpu as pltpu
from jax.experimental.pallas import tpu_sc as plsc

N = 10000
E = 320000
D = 128
HW = 64
CD = 16
NC, NS = 2, 16
NW = NC * NS
C = 128
NSLOT = 4

EPW = E // NW
NCH_R = 80
EPT = E // NS
NCH_C = 160

NPAD = 10008
RPS = 624
_CHUNKS = [(0, 128), (128, 128), (256, 128), (384, 128), (512, 112)]
_TAIL_ROWS = (NS * RPS, N - NS * RPS)
DEGW = 16


def _zero_vmem(buf, nrows, width):
    @pl.loop(0, nrows)
    def _(i):
        for j in range(width // 16):
            buf[i, pl.ds(j * 16, 16)] = jnp.zeros((16,), jnp.float32)


def _zero_acc_slice(sid, zsrc, acc):
    r0 = sid * RPS
    for off, m in _CHUNKS:
        pltpu.sync_copy(zsrc.at[pl.ds(0, m)], acc.at[pl.ds(r0 + off, m)])

    @pl.when(sid == NS - 1)
    def _():
        t0, tm = _TAIL_ROWS
        pltpu.sync_copy(zsrc.at[pl.ds(0, tm)], acc.at[pl.ds(t0, tm)])


def _writeback(sid, cid, acc, bounce, out_hbm):
    r0 = sid * RPS
    for off, m in _CHUNKS:
        rr = r0 + off
        pltpu.sync_copy(acc.at[pl.ds(rr, m)], bounce.at[pl.ds(0, m)])
        pltpu.sync_copy(bounce.at[pl.ds(0, m)], out_hbm.at[cid, pl.ds(rr, m)])

    @pl.when(sid == NS - 1)
    def _():
        t0, tm = _TAIL_ROWS
        pltpu.sync_copy(acc.at[pl.ds(t0, tm)], bounce.at[pl.ds(0, tm)])
        pltpu.sync_copy(bounce.at[pl.ds(0, tm)], out_hbm.at[cid, pl.ds(t0, tm)])


def _deg_body(dst3_hbm, deg_hbm, acc, dstall, ones, bounce, s0, s1, s2, s3):
    cid = lax.axis_index("c")
    sid = lax.axis_index("s")
    wid = cid * NS + sid
    sems = [s0, s1, s2, s3]

    @pl.loop(0, C)
    def _(i):
        ones[i, pl.ds(0, 16)] = jnp.ones((16,), jnp.float32)
    _zero_vmem(bounce, C, DEGW)
    _zero_acc_slice(sid, bounce, acc)
    pltpu.sync_copy(dst3_hbm.at[wid], dstall)
    plsc.subcore_barrier()

    for b in range(NSLOT):
        pltpu.async_copy(ones, acc.at[dstall.at[b]], sems[b], add=True)

    @pl.loop(0, NCH_R // NSLOT)
    def _(g):
        for b in range(NSLOT):
            i = g * NSLOT + b
            pltpu.make_async_copy(ones, acc.at[dstall.at[i]], sems[b]).wait()

            @pl.when(g < NCH_R // NSLOT - 1)
            def _():
                pltpu.async_copy(ones, acc.at[dstall.at[i + NSLOT]], sems[b],
                                 add=True)

    plsc.subcore_barrier()
    _writeback(sid, cid, acc, bounce, deg_hbm)


def _cprop_body(yc_hbm, src3_hbm, dst3_hbm, pc_hbm, acc, srcall, dstall,
                r0_, r1_, r2_, r3_, g0, g1, g2, g3, s0, s1, s2, s3):
    cid = lax.axis_index("c")
    sid = lax.axis_index("s")
    wid = cid * NS + sid
    rows = [r0_, r1_, r2_, r3_]
    gsem = [g0, g1, g2, g3]
    ssem = [s0, s1, s2, s3]

    _zero_vmem(r0_, C, CD)
    _zero_acc_slice(sid, r0_, acc)
    pltpu.sync_copy(src3_hbm.at[wid], srcall)
    pltpu.sync_copy(dst3_hbm.at[wid], dstall)
    plsc.subcore_barrier()

    for b in range(NSLOT):
        pltpu.async_copy(yc_hbm.at[srcall.at[b]], rows[b], gsem[b])

    @pl.loop(0, NCH_R // NSLOT)
    def _(g):
        for b in range(NSLOT):
            i = g * NSLOT + b
            pltpu.make_async_copy(yc_hbm.at[srcall.at[i]], rows[b],
                                  gsem[b]).wait()
            pltpu.async_copy(rows[b], acc.at[dstall.at[i]], ssem[b],
                             add=True).wait()

            @pl.when(g < NCH_R // NSLOT - 1)
            def _():
                pltpu.async_copy(yc_hbm.at[srcall.at[i + NSLOT]], rows[b],
                                 gsem[b])

    plsc.subcore_barrier()
    _writeback(sid, cid, acc, r0_, pc_hbm)


def _colprop_body(y_hbm, srcT_hbm, dstT_hbm, p_hbm, acc, srcall, dstall,
                  r0_, r1_, r2_, r3_, g0, g1, g2, g3, s0, s1, s2, s3):
    cid = lax.axis_index("c")
    sid = lax.axis_index("s")
    wid = cid * NS + sid
    rows = [r0_, r1_, r2_, r3_]
    gsem = [g0, g1, g2, g3]
    ssem = [s0, s1, s2, s3]

    _zero_vmem(r0_, C, HW)
    _zero_acc_slice(sid, r0_, acc)
    pltpu.sync_copy(srcT_hbm.at[wid], srcall)
    pltpu.sync_copy(dstT_hbm.at[sid], dstall)
    plsc.subcore_barrier()

    def gref(i):
        return y_hbm.at[srcall.at[i]]

    for b in range(NSLOT):
        pltpu.async_copy(gref(b), rows[b], gsem[b])

    @pl.loop(0, NCH_C // NSLOT)
    def _(g):
        for b in range(NSLOT):
            i = g * NSLOT + b
            pltpu.make_async_copy(gref(i), rows[b], gsem[b]).wait()
            pltpu.async_copy(rows[b], acc.at[dstall.at[i]], ssem[b],
                             add=True).wait()

            @pl.when(g < NCH_C // NSLOT - 1)
            def _():
                pltpu.async_copy(gref(i + NSLOT), rows[b], gsem[b])

    plsc.subcore_barrier()
    _writeback(sid, cid, acc, r0_, p_hbm)


@functools.cache
def _sc_mesh():
    return plsc.VectorSubcoreMesh(core_axis_name="c", subcore_axis_name="s",
                                  num_cores=NC, num_subcores=NS)


_SC_PARAMS = None


def _sc_params():
    return pltpu.CompilerParams(use_tc_tiling_on_sc=False)


def _sems(n):
    return [pltpu.SemaphoreType.DMA] * n


@jax.jit
def _sc_degree(dst3):
    return pl.kernel(
        _deg_body,
        out_type=jax.ShapeDtypeStruct((NC, N, DEGW), jnp.float32),
        mesh=_sc_mesh(),
        compiler_params=_sc_params(),
        scratch_types=[
            pltpu.VMEM_SHARED((NPAD, DEGW), jnp.float32),
            pltpu.VMEM((NCH_R, C), jnp.int32),
            pltpu.VMEM((C, DEGW), jnp.float32),
            pltpu.VMEM((C, DEGW), jnp.float32),
        ] + _sems(NSLOT),
    )(dst3)


@jax.jit
def _sc_cprop(yc, src3, dst3):
    return pl.kernel(
        _cprop_body,
        out_type=jax.ShapeDtypeStruct((NC, N, CD), jnp.float32),
        mesh=_sc_mesh(),
        compiler_params=_sc_params(),
        scratch_types=[
            pltpu.VMEM_SHARED((NPAD, CD), jnp.float32),
            pltpu.VMEM((NCH_R, C), jnp.int32),
            pltpu.VMEM((NCH_R, C), jnp.int32),
        ] + [pltpu.VMEM((C, CD), jnp.float32)] * NSLOT
          + _sems(2 * NSLOT),
    )(yc, src3, dst3)


@jax.jit
def _sc_colprop(y, srcT2, dstT):
    return pl.kernel(
        _colprop_body,
        out_type=jax.ShapeDtypeStruct((NC, N, HW), jnp.float32),
        mesh=_sc_mesh(),
        compiler_params=_sc_params(),
        scratch_types=[
            pltpu.VMEM_SHARED((NPAD, HW), jnp.float32),
            pltpu.VMEM((NCH_C, C), jnp.int32),
            pltpu.VMEM((NCH_C, C), jnp.int32),
        ] + [pltpu.VMEM((C, HW), jnp.float32)] * NSLOT
          + _sems(2 * NSLOT),
    )(y.reshape(2 * N, HW), srcT2, dstT)


_RB = 2000
_GRID = (N // _RB,)


def _tc_call(body, out_shapes, in_specs, out_specs):
    return pl.pallas_call(
        body,
        grid=_GRID,
        in_specs=in_specs,
        out_specs=out_specs,
        out_shape=out_shapes,
    )


def _rows(w):
    return pl.BlockSpec((_RB, w), lambda i: (i, 0))


def _split(w):
    return pl.BlockSpec((NC, _RB, w), lambda i: (0, i, 0))


def _full(a, b):
    return pl.BlockSpec((a, b), lambda i: (0, 0))


def _cat(p):
    return jnp.concatenate([p[0], p[1]], axis=1)


def _k1_body(degp, f, c, wf, dis_o, yf_o, yc_o):
    deg = degp[0, :, 0:1] + degp[1, :, 0:1] + 1.0
    dis = lax.rsqrt(deg)
    dis_o[...] = dis
    yf_o[...] = dis * jnp.dot(f[...], wf[...],
                              preferred_element_type=jnp.float32)
    yc_o[...] = dis * c[...]


def _k2_body(pf, yf, bf, pc, yc, bc, dis, wc, wh1, wh2, yh_o):
    d = dis[...]
    f2h = jnp.tanh(d * (_cat(pf) + yf[...]) + bf[...])
    ac = d * (pc[0] + pc[1] + yc[...])
    c2h = jnp.tanh(jnp.dot(ac, wc[...], preferred_element_type=jnp.float32)
                   + bc[...])
    yh_o[...] = d * (jnp.dot(f2h, wh1[...], preferred_element_type=jnp.float32)
                     + jnp.dot(c2h, wh2[...],
                               preferred_element_type=jnp.float32))


def _k3_body(ph, yh, bh, dis, y2_o):
    d = dis[...]
    h2 = jnp.tanh(d * (_cat(ph) + yh[...]) + bh[...])
    y2_o[...] = d * h2


def _k4_body(p2, y2, dis, wm, bm, wv, bv, noise, z_o, mean_o, logvar_o):
    ah = dis[...] * (_cat(p2) + y2[...])
    mean = jnp.dot(ah, wm[...], preferred_element_type=jnp.float32) + bm[...]
    logvar = jnp.dot(ah, wv[...], preferred_element_type=jnp.float32) + bv[...]
    mean_o[...] = mean
    logvar_o[...] = logvar
    z_o[...] = noise[...] * jnp.exp(0.5 * logvar) + mean


def _pad_pages(x, nper, nch, fill):
    ng = E // nper
    pad = jnp.full((ng, nch * C - nper), fill, jnp.int32)
    return jnp.concatenate([x.reshape(ng, nper), pad], axis=1).reshape(
        ng, nch, C)


def kernel(feature, condition, edge_index, W_f, b_f, W_c, b_c, W_h, b_h,
           W_m, b_m, W_v, b_v):
    src = edge_index[0].astype(jnp.int32)
    dst = edge_index[1].astype(jnp.int32)
    ldim = W_m.shape[1]

    src3 = _pad_pages(src, EPW, NCH_R, 0)
    dst3 = _pad_pages(dst, EPW, NCH_R, N)
    srcT_b = _pad_pages(src * 2, EPT, NCH_C, 0)
    srcT2 = jnp.concatenate([srcT_b, srcT_b + 1], axis=0)
    dstT = _pad_pages(dst, EPT, NCH_C, N)

    degp = _sc_degree(dst3)

    k1 = _tc_call(
        _k1_body,
        (jax.ShapeDtypeStruct((N, 1), jnp.float32),
         jax.ShapeDtypeStruct((N, D), jnp.float32),
         jax.ShapeDtypeStruct((N, CD), jnp.float32)),
        [_split(DEGW), _rows(D), _rows(CD), _full(D, D)],
        (_rows(1), _rows(D), _rows(CD)),
    )
    dis, yf, yc = k1(degp, feature, condition, W_f)

    pc = _sc_cprop(yc, src3, dst3)
    pf = _sc_colprop(yf, srcT2, dstT)

    b2 = lambda b: b.reshape(1, -1)
    k2 = _tc_call(
        _k2_body,
        jax.ShapeDtypeStruct((N, D), jnp.float32),
        [_split(HW), _rows(D), _full(1, D), _split(CD), _rows(CD),
         _full(1, D), _rows(1), _full(CD, D), _full(D, D), _full(D, D)],
        _rows(D),
    )
    yh = k2(pf, yf, b2(b_f), pc, yc, b2(b_c), dis, W_c, W_h[:D], W_h[D:])

    ph = _sc_colprop(yh, srcT2, dstT)

    k3 = _tc_call(
        _k3_body,
        jax.ShapeDtypeStruct((N, D), jnp.float32),
        [_split(HW), _rows(D), _full(1, D), _rows(1)],
        _rows(D),
    )
    y2 = k3(ph, yh, b2(b_h), dis)

    p2 = _sc_colprop(y2, srcT2, dstT)

    noise = jax.random.normal(jax.random.key(42), (N, ldim), jnp.float32)
    k4 = _tc_call(
        _k4_body,
        (jax.ShapeDtypeStruct((N, ldim), jnp.float32),
         jax.ShapeDtypeStruct((N, ldim), jnp.float32),
         jax.ShapeDtypeStruct((N, ldim), jnp.float32)),
        [_split(HW), _rows(D), _rows(1), _full(D, ldim), _full(1, ldim),
         _full(D, ldim), _full(1, ldim), _rows(ldim)],
        (_rows(ldim), _rows(ldim), _rows(ldim)),
    )
    z, mean, logvar = k4(p2, y2, dis, W_m, b2(b_m), W_v, b2(b_v), noise)
    return (z, mean, logvar)


# speedup vs baseline: 1.7062x; 1.7062x over previous
import functools

import jax
import jax.numpy as jnp
from jax import lax
from jax.experimental import pallas as pl
from jax.experimental.pallas import tpu as pltpu
from jax.experimental.pallas import tpu_sc as plsc

N = 10000
E = 320000
D = 128
HW = 64
CD = 16
NC, NS = 2, 16
NW = NC * NS
C = 128
NSLOT = 4

EPW = E // NW
NCH_R = 80
CW = 112
NCH_W = 90
NSLOT_W = 2

NPAD = 10008
RPS = 624
_TAIL_ROWS = (NS * RPS, N - NS * RPS)


def _blocks(total, step):
    out, off = [], 0
    while off < total:
        out.append((off, min(step, total - off)))
        off += step
    return out
DEGW = 16


def _zero_vmem(buf, nrows, width):
    @pl.loop(0, nrows)
    def _(i):
        for j in range(width // 16):
            buf[i, pl.ds(j * 16, 16)] = jnp.zeros((16,), jnp.float32)


def _zero_acc_slice(sid, zsrc, acc, step=128):
    r0 = sid * RPS
    for off, m in _blocks(RPS, step):
        pltpu.sync_copy(zsrc.at[pl.ds(0, m)], acc.at[pl.ds(r0 + off, m)])

    @pl.when(sid == NS - 1)
    def _():
        t0, tm = _TAIL_ROWS
        pltpu.sync_copy(zsrc.at[pl.ds(0, tm)], acc.at[pl.ds(t0, tm)])


def _writeback(sid, cid, acc, bounce, out_hbm, step=128):
    r0 = sid * RPS
    for off, m in _blocks(RPS, step):
        rr = r0 + off
        pltpu.sync_copy(acc.at[pl.ds(rr, m)], bounce.at[pl.ds(0, m)])
        pltpu.sync_copy(bounce.at[pl.ds(0, m)], out_hbm.at[cid, pl.ds(rr, m)])

    @pl.when(sid == NS - 1)
    def _():
        t0, tm = _TAIL_ROWS
        pltpu.sync_copy(acc.at[pl.ds(t0, tm)], bounce.at[pl.ds(0, tm)])
        pltpu.sync_copy(bounce.at[pl.ds(0, tm)], out_hbm.at[cid, pl.ds(t0, tm)])


def _deg_body(dst3_hbm, deg_hbm, acc, dstall, ones, bounce, s0, s1, s2, s3):
    cid = lax.axis_index("c")
    sid = lax.axis_index("s")
    wid = cid * NS + sid
    sems = [s0, s1, s2, s3]

    @pl.loop(0, C)
    def _(i):
        ones[i, pl.ds(0, 16)] = jnp.ones((16,), jnp.float32)
    _zero_vmem(bounce, C, DEGW)
    _zero_acc_slice(sid, bounce, acc)
    pltpu.sync_copy(dst3_hbm.at[wid], dstall)
    plsc.subcore_barrier()

    for b in range(NSLOT):
        pltpu.async_copy(ones, acc.at[dstall.at[b]], sems[b], add=True)

    @pl.loop(0, NCH_R // NSLOT)
    def _(g):
        for b in range(NSLOT):
            i = g * NSLOT + b
            pltpu.make_async_copy(ones, acc.at[dstall.at[i]], sems[b]).wait()

            @pl.when(g < NCH_R // NSLOT - 1)
            def _():
                pltpu.async_copy(ones, acc.at[dstall.at[i + NSLOT]], sems[b],
                                 add=True)

    plsc.subcore_barrier()
    _writeback(sid, cid, acc, bounce, deg_hbm)


def _make_prop_body(width, c_, nch, nslot):
    def body(y_hbm, src3_hbm, dst3_hbm, p_hbm, acc, srcall, dstall, *bufs):
        cid = lax.axis_index("c")
        sid = lax.axis_index("s")
        wid = cid * NS + sid
        rows = list(bufs[:nslot])
        gsem = list(bufs[nslot:2 * nslot])
        ssem = list(bufs[2 * nslot:3 * nslot])

        _zero_vmem(rows[0], c_, width)
        _zero_acc_slice(sid, rows[0], acc, c_)
        pltpu.sync_copy(src3_hbm.at[wid], srcall)
        pltpu.sync_copy(dst3_hbm.at[wid], dstall)
        plsc.subcore_barrier()

        for b in range(nslot):
            pltpu.async_copy(y_hbm.at[srcall.at[b]], rows[b], gsem[b])

        @pl.loop(0, nch // nslot)
        def _(g):
            for b in range(nslot):
                i = g * nslot + b
                pltpu.make_async_copy(y_hbm.at[srcall.at[i]], rows[b],
                                      gsem[b]).wait()
                pltpu.async_copy(rows[b], acc.at[dstall.at[i]], ssem[b],
                                 add=True).wait()

                @pl.when(g < nch // nslot - 1)
                def _():
                    pltpu.async_copy(y_hbm.at[srcall.at[i + nslot]], rows[b],
                                     gsem[b])

        plsc.subcore_barrier()
        _writeback(sid, cid, acc, rows[0], p_hbm, c_)
    return body


@functools.cache
def _sc_mesh():
    return plsc.VectorSubcoreMesh(core_axis_name="c", subcore_axis_name="s",
                                  num_cores=NC, num_subcores=NS)


_SC_PARAMS = None


def _sc_params():
    return pltpu.CompilerParams(use_tc_tiling_on_sc=False)


def _sems(n):
    return [pltpu.SemaphoreType.DMA] * n


@jax.jit
def _sc_degree(dst3):
    return pl.kernel(
        _deg_body,
        out_type=jax.ShapeDtypeStruct((NC, N, DEGW), jnp.float32),
        mesh=_sc_mesh(),
        compiler_params=_sc_params(),
        scratch_types=[
            pltpu.VMEM_SHARED((NPAD, DEGW), jnp.float32),
            pltpu.VMEM((NCH_R, C), jnp.int32),
            pltpu.VMEM((C, DEGW), jnp.float32),
            pltpu.VMEM((C, DEGW), jnp.float32),
        ] + _sems(NSLOT),
    )(dst3)


@jax.jit
def _sc_cprop(yc, src3, dst3):
    return pl.kernel(
        _make_prop_body(CD, C, NCH_R, NSLOT),
        out_type=jax.ShapeDtypeStruct((NC, N, CD), jnp.float32),
        mesh=_sc_mesh(),
        compiler_params=_sc_params(),
        scratch_types=[
            pltpu.VMEM_SHARED((NPAD, CD), jnp.float32),
            pltpu.VMEM((NCH_R, C), jnp.int32),
            pltpu.VMEM((NCH_R, C), jnp.int32),
        ] + [pltpu.VMEM((C, CD), jnp.float32)] * NSLOT
          + _sems(2 * NSLOT),
    )(yc, src3, dst3)


@jax.jit
def _sc_prop(y, srcW, dstW):
    return pl.kernel(
        _make_prop_body(D, CW, NCH_W, NSLOT_W),
        out_type=jax.ShapeDtypeStruct((NC, N, D), jnp.float32),
        mesh=_sc_mesh(),
        compiler_params=_sc_params(),
        scratch_types=[
            pltpu.VMEM_SHARED((NPAD, D), jnp.float32),
            pltpu.VMEM((NCH_W, CW), jnp.int32),
            pltpu.VMEM((NCH_W, CW), jnp.int32),
        ] + [pltpu.VMEM((CW, D), jnp.float32)] * NSLOT_W
          + _sems(2 * NSLOT_W),
    )(y, srcW, dstW)


_RB = 2000
_GRID = (N // _RB,)


def _tc_call(body, out_shapes, in_specs, out_specs):
    return pl.pallas_call(
        body,
        grid=_GRID,
        in_specs=in_specs,
        out_specs=out_specs,
        out_shape=out_shapes,
    )


def _rows(w):
    return pl.BlockSpec((_RB, w), lambda i: (i, 0))


def _split(w):
    return pl.BlockSpec((NC, _RB, w), lambda i: (0, i, 0))


def _full(a, b):
    return pl.BlockSpec((a, b), lambda i: (0, 0))


def _psum(p):
    return p[0] + p[1]


def _k1_body(degp, f, c, wf, dis_o, yf_o, yc_o):
    deg = degp[0, :, 0:1] + degp[1, :, 0:1] + 1.0
    dis = lax.rsqrt(deg)
    dis_o[...] = dis
    yf_o[...] = dis * jnp.dot(f[...], wf[...],
                              preferred_element_type=jnp.float32)
    yc_o[...] = dis * c[...]


def _k2_body(pf, yf, bf, pc, yc, bc, dis, wc, wh1, wh2, yh_o):
    d = dis[...]
    f2h = jnp.tanh(d * (_psum(pf) + yf[...]) + bf[...])
    ac = d * (pc[0] + pc[1] + yc[...])
    c2h = jnp.tanh(jnp.dot(ac, wc[...], preferred_element_type=jnp.float32)
                   + bc[...])
    yh_o[...] = d * (jnp.dot(f2h, wh1[...], preferred_element_type=jnp.float32)
                     + jnp.dot(c2h, wh2[...],
                               preferred_element_type=jnp.float32))


def _k3_body(ph, yh, bh, dis, y2_o):
    d = dis[...]
    h2 = jnp.tanh(d * (_psum(ph) + yh[...]) + bh[...])
    y2_o[...] = d * h2


def _k4_body(p2, y2, dis, wm, bm, wv, bv, noise, z_o, mean_o, logvar_o):
    ah = dis[...] * (_psum(p2) + y2[...])
    mean = jnp.dot(ah, wm[...], preferred_element_type=jnp.float32) + bm[...]
    logvar = jnp.dot(ah, wv[...], preferred_element_type=jnp.float32) + bv[...]
    mean_o[...] = mean
    logvar_o[...] = logvar
    z_o[...] = noise[...] * jnp.exp(0.5 * logvar) + mean


def _pad_pages(x, nper, nch, fill, c_):
    ng = E // nper
    pad = jnp.full((ng, nch * c_ - nper), fill, jnp.int32)
    return jnp.concatenate([x.reshape(ng, nper), pad], axis=1).reshape(
        ng, nch, c_)


def kernel(feature, condition, edge_index, W_f, b_f, W_c, b_c, W_h, b_h,
           W_m, b_m, W_v, b_v):
    src = edge_index[0].astype(jnp.int32)
    dst = edge_index[1].astype(jnp.int32)
    ldim = W_m.shape[1]

    src3 = _pad_pages(src, EPW, NCH_R, 0, C)
    dst3 = _pad_pages(dst, EPW, NCH_R, N, C)
    srcW = _pad_pages(src, EPW, NCH_W, 0, CW)
    dstW = _pad_pages(dst, EPW, NCH_W, N, CW)

    degp = _sc_degree(dst3)

    k1 = _tc_call(
        _k1_body,
        (jax.ShapeDtypeStruct((N, 1), jnp.float32),
         jax.ShapeDtypeStruct((N, D), jnp.float32),
         jax.ShapeDtypeStruct((N, CD), jnp.float32)),
        [_split(DEGW), _rows(D), _rows(CD), _full(D, D)],
        (_rows(1), _rows(D), _rows(CD)),
    )
    dis, yf, yc = k1(degp, feature, condition, W_f)

    pc = _sc_cprop(yc, src3, dst3)
    pf = _sc_prop(yf, srcW, dstW)

    b2 = lambda b: b.reshape(1, -1)
    k2 = _tc_call(
        _k2_body,
        jax.ShapeDtypeStruct((N, D), jnp.float32),
        [_split(D), _rows(D), _full(1, D), _split(CD), _rows(CD),
         _full(1, D), _rows(1), _full(CD, D), _full(D, D), _full(D, D)],
        _rows(D),
    )
    yh = k2(pf, yf, b2(b_f), pc, yc, b2(b_c), dis, W_c, W_h[:D], W_h[D:])

    ph = _sc_prop(yh, srcW, dstW)

    k3 = _tc_call(
        _k3_body,
        jax.ShapeDtypeStruct((N, D), jnp.float32),
        [_split(D), _rows(D), _full(1, D), _rows(1)],
        _rows(D),
    )
    y2 = k3(ph, yh, b2(b_h), dis)

    p2 = _sc_prop(y2, srcW, dstW)

    noise = jax.random.normal(jax.random.key(42), (N, ldim), jnp.float32)
    k4 = _tc_call(
        _k4_body,
        (jax.ShapeDtypeStruct((N, ldim), jnp.float32),
         jax.ShapeDtypeStruct((N, ldim), jnp.float32),
         jax.ShapeDtypeStruct((N, ldim), jnp.float32)),
        [_split(D), _rows(D), _rows(1), _full(D, ldim), _full(1, ldim),
         _full(D, ldim), _full(1, ldim), _rows(ldim)],
        (_rows(ldim), _rows(ldim), _rows(ldim)),
    )
    z, mean, logvar = k4(p2, y2, dis, W_m, b2(b_m), W_v, b2(b_v), noise)
    return (z, mean, logvar)
